# Initial kernel scaffold; baseline (speedup 1.0000x reference)
import jax
import jax.numpy as jnp
from jax.experimental import pallas as pl


def kernel(inputs):
    raise NotImplementedError("write your pallas kernel here")


# speedup vs baseline: 343.7342x; 343.7342x over previous
import functools

import numpy as np
import jax
import jax.numpy as jnp
from jax import lax
from jax.experimental import pallas as pl

_H = 224
_K = 17
_S = 4
_NW = 52
_L = 5
_G_MEAN = 85.384
_G_STD = 53.798
_THRESH = (0.5, _G_MEAN - _G_STD, _G_MEAN, _G_MEAN + _G_STD)

_OFFSETS = [(0, 1), (1, 1), (1, 0), (1, -1),
            (0, 2), (1, 1), (2, 0), (1, -1),
            (0, 4), (3, 3), (4, 0), (3, -3)]
_UNIQUE = []
for _o in _OFFSETS:
    if _o not in _UNIQUE:
        _UNIQUE.append(_o)
_T2U = [_UNIQUE.index(_o) for _o in _OFFSETS]
_NU = len(_UNIQUE)


def _build_mats():
    r = np.arange(_H)[None, :]
    k4 = (_S * np.arange(_NW))[:, None]
    mats = []
    for dr, _dc in _UNIQUE:
        mats.append(((r >= k4) & (r < k4 + _K - dr)).astype(np.float32))
    for _dr, dc in _UNIQUE:
        lo, hi = max(0, -dc), max(0, dc)
        mats.append(((r >= k4 + lo) & (r < k4 + _K - hi)).astype(np.float32))
    mats.append(((r >= k4) & (r < k4 + _K)).astype(np.float32))
    sel = np.zeros((_NW, _H), np.float32)
    sel[np.arange(_NW), _S * np.arange(_NW)] = 1.0
    mats.append(sel)
    return np.stack(mats)


def _dot(a, b):
    return lax.dot_general(a, b, (((1,), (0,)), ((), ())),
                           precision=lax.Precision.HIGHEST,
                           preferred_element_type=jnp.float32)


def _dot_nt(a, b):
    return lax.dot_general(a, b, (((1,), (1,)), ((), ())),
                           precision=lax.Precision.HIGHEST,
                           preferred_element_type=jnp.float32)


def _shift(m, dr, dc, fill):
    src = m[dr:, max(dc, 0):_H + min(dc, 0)]
    rows = _H - dr
    if dc > 0:
        src = jnp.concatenate(
            [src, jnp.full((rows, dc), fill, m.dtype)], axis=1)
    elif dc < 0:
        src = jnp.concatenate(
            [jnp.full((rows, -dc), fill, m.dtype), src], axis=1)
    if dr > 0:
        src = jnp.concatenate(
            [src, jnp.full((dr, _H), fill, m.dtype)], axis=0)
    return src


def _body(x_ref, mats_ref, out_ref):
    x = x_ref[0, 0]
    m17 = mats_ref[2 * _NU]
    sel = mats_ref[2 * _NU + 1][:, :_H - _K + 1]

    npx = float(_K * _K)
    s1 = _dot_nt(_dot(m17, x), m17)
    s2 = _dot_nt(_dot(m17, x * x), m17)
    mean = s1 / npx
    var = jnp.maximum(s2 / npx - mean * mean, 0.0)
    std = jnp.sqrt(var)

    def box_red(m, op):
        cm = m[:, 0:_H - _K + 1]
        for i in range(1, _K):
            cm = op(cm, m[:, i:_H - _K + 1 + i])
        csel = _dot_nt(cm, sel)
        rm = csel[0:_H - _K + 1, :]
        for i in range(1, _K):
            rm = op(rm, csel[i:_H - _K + 1 + i, :])
        return _dot(sel, rm)

    wmax = box_red(x, jnp.maximum)
    wmin = box_red(x, jnp.minimum)

    means_n = mean / _G_MEAN
    std_n = std / _G_STD
    mx_n = (wmax - means_n) / _G_STD
    mn_n = (means_n - wmin) / _G_STD

    qlev = jnp.zeros_like(x)
    for t in _THRESH:
        qlev = qlev + (x >= t).astype(jnp.float32)
    onehot = [(qlev == float(a)).astype(jnp.float32) for a in range(_L)]

    hc = [_dot_nt(_dot(m17, onehot[a]), m17) for a in range(1, _L)]
    hs = hc[0] + hc[1] + hc[2] + hc[3]
    hsg = jnp.where(hs == 0.0, 1.0, hs)
    hist = [h / hsg for h in hc]

    uniq_feats = []
    for u, (dr, dc) in enumerate(_UNIQUE):
        ar = mats_ref[u]
        ac = mats_ref[_NU + u]
        qs = _shift(qlev, dr, dc, -1.0)
        sh = [(qs == float(b)).astype(jnp.float32) for b in range(_L)]
        cnt = [[_dot_nt(_dot(ar, onehot[a] * sh[b]), ac)
                for b in range(_L)] for a in range(_L)]
        inv = 1.0 / float(2 * (_K - dr) * (_K - abs(dc)))
        N = [[(cnt[a][b] + cnt[b][a]) * inv for b in range(_L)]
             for a in range(_L)]
        con = jnp.zeros_like(N[0][0])
        hom = jnp.zeros_like(con)
        ene2 = jnp.zeros_like(con)
        ent = jnp.zeros_like(con)
        mi = jnp.zeros_like(con)
        mj = jnp.zeros_like(con)
        rowsum = [jnp.zeros_like(con) for _ in range(_L)]
        colsum = [jnp.zeros_like(con) for _ in range(_L)]
        for a in range(_L):
            for b in range(_L):
                nab = N[a][b]
                d2 = float((a - b) ** 2)
                if d2:
                    con = con + d2 * nab
                hom = hom + nab * (1.0 / (1.0 + d2))
                ene2 = ene2 + nab * nab
                ent = ent - nab * (jnp.log2(nab + 1e-8))
                rowsum[a] = rowsum[a] + nab
                colsum[b] = colsum[b] + nab
        for a in range(1, _L):
            mi = mi + float(a) * rowsum[a]
            mj = mj + float(a) * colsum[a]
        vi = jnp.zeros_like(con)
        vj = jnp.zeros_like(con)
        cov = jnp.zeros_like(con)
        for a in range(_L):
            da = float(a) - mi
            vi = vi + da * da * rowsum[a]
            db = float(a) - mj
            vj = vj + db * db * colsum[a]
        for a in range(_L):
            da = float(a) - mi
            for b in range(_L):
                cov = cov + da * (float(b) - mj) * N[a][b]
        stdi = jnp.sqrt(vi)
        stdj = jnp.sqrt(vj)
        den = stdi * stdj
        corr = jnp.where((stdi < 1e-15) | (stdj < 1e-15), 1.0,
                         cov / jnp.where(den == 0.0, 1.0, den))
        energy = jnp.sqrt(ene2)
        uniq_feats.append(tuple(jnp.sum(m)
                                for m in (con, hom, energy, corr, ent)))

    inv_w = 1.0 / float(_NW * _NW)
    vals = [jnp.sum(means_n), jnp.sum(std_n), jnp.sum(mx_n), jnp.sum(mn_n)]
    vals += [jnp.sum(h) for h in hist]
    for f in range(5):
        vals += [uniq_feats[_T2U[t]][f] for t in range(len(_OFFSETS))]
    row = jnp.concatenate([(v * inv_w).reshape(1, 1) for v in vals], axis=1)
    out_ref[...] = row[None]


def kernel(x):
    b = x.shape[0]
    mats = jnp.asarray(_build_mats())
    return pl.pallas_call(
        _body,
        grid=(b,),
        in_specs=[
            pl.BlockSpec((1, 1, _H, _H), lambda i: (i, 0, 0, 0)),
            pl.BlockSpec(mats.shape, lambda i: (0, 0, 0)),
        ],
        out_specs=pl.BlockSpec((1, 1, 68), lambda i: (i, 0, 0)),
        out_shape=jax.ShapeDtypeStruct((b, 1, 68), jnp.float32),
    )(x, mats).reshape(b, 68)
